# baseline (speedup 1.0000x reference)
import jax
import jax.numpy as jnp
from jax import lax
from jax.experimental import pallas as pl
from jax.experimental.pallas import tpu as pltpu
from jax.experimental.pallas import tpu_sc as plsc

N = 10000
E0 = 320000
E = E0 + N
D = 128
HEADS = 8
NC, NS, L = 2, 16, 16
NW = NC * NS
B = 128
CH = 82
EPAD = NW * B * CH
CH0 = 80
EPAD0 = NW * B * CH0
RPT = 632
NPAD = NS * RPT
PACK = 16384


def _ln(x, g, b):
    m = jnp.mean(x, axis=-1, keepdims=True)
    c = x - m
    v = jnp.mean(c * c, axis=-1, keepdims=True)
    return c * lax.rsqrt(v + 1e-5) * g + b


def _elu(x):
    return jnp.where(x > 0, x, jnp.exp(jnp.minimum(x, 0.0)) - 1.0)


def _tc_in_body(x_ref, win, bin_, gin, btin, w1, as_, ad_, hw_ref, asr_ref, adt_ref):
    h0 = jnp.dot(x_ref[...], win[...], preferred_element_type=jnp.float32) + bin_[...]
    h = _ln(h0, gin[...], btin[...])
    hw = jnp.dot(h, w1[...], preferred_element_type=jnp.float32)
    hw_ref[...] = hw
    asr_ref[...] = jnp.dot(hw, as_[...], preferred_element_type=jnp.float32)
    adt_ref[...] = jnp.dot(hw, ad_[...], preferred_element_type=jnp.float32)


def _tc_mid_body(numa_ref, numb_ref, den_ref, p_ref, b_ref, g_ref, bt_ref,
                 w2, as_, ad_, hw_ref, asr_ref, adt_ref):
    num = jnp.concatenate([numa_ref[0] + numa_ref[1],
                           numb_ref[0] + numb_ref[1]], axis=-1)
    den = den_ref[0] + den_ref[1]
    denf = jnp.dot(den, p_ref[...], preferred_element_type=jnp.float32)
    gat = num / denf + b_ref[...]
    h = _elu(_ln(gat, g_ref[...], bt_ref[...]))
    hw = jnp.dot(h, w2[...], preferred_element_type=jnp.float32)
    hw_ref[...] = hw
    asr_ref[...] = jnp.dot(hw, as_[...], preferred_element_type=jnp.float32)
    adt_ref[...] = jnp.dot(hw, ad_[...], preferred_element_type=jnp.float32)


def _tc_out_body(numa_ref, numb_ref, den_ref, p_ref, b_ref, g_ref, bt_ref,
                 we1, be1, we2p, be2p, wra, wrb,
                 ent_ref, ha_ref, hb_ref):
    num = jnp.concatenate([numa_ref[0] + numa_ref[1],
                           numb_ref[0] + numb_ref[1]], axis=-1)
    den = den_ref[0] + den_ref[1]
    denf = jnp.dot(den, p_ref[...], preferred_element_type=jnp.float32)
    gat = num / denf + b_ref[...]
    h = _elu(_ln(gat, g_ref[...], bt_ref[...]))
    t = jnp.maximum(jnp.dot(h, we1[...], preferred_element_type=jnp.float32) + be1[...], 0.0)
    logits = jnp.dot(t, we2p[...], preferred_element_type=jnp.float32) + be2p[...]
    m = jnp.max(logits, axis=-1, keepdims=True)
    ex = jnp.exp(logits - m)
    sm = ex / jnp.sum(ex, axis=-1, keepdims=True)
    ent_ref[...] = jnp.log(sm + 1e-8)
    ha_ref[...] = jnp.dot(h, wra[...], preferred_element_type=jnp.float32)
    hb_ref[...] = jnp.dot(h, wrb[...], preferred_element_type=jnp.float32)


def _tc_rel_body(g_ref, br1_ref, wr2_ref, br2_ref, rel_ref):
    t = jnp.maximum(g_ref[...] + br1_ref[...], 0.0)
    rel_ref[...] = jnp.dot(t, wr2_ref[...], preferred_element_type=jnp.float32) + br2_ref[...]


def _full(shape):
    return pl.BlockSpec(shape, lambda i: tuple(0 for _ in shape))


def _splat(v, lane):
    return lax.gather(
        v, jnp.full((L, 1), lane, jnp.int32),
        dimension_numbers=lax.GatherDimensionNumbers(
            offset_dims=(), collapsed_slice_dims=(0,), start_index_map=(0,)),
        slice_sizes=(1,),
        mode=lax.GatherScatterMode.PROMISE_IN_BOUNDS)


def _unpack_idx(idx_c, c, isb, idb):
    for k in range(B // L):
        v = idx_c[c, pl.ds(k * L, L)]
        isb[pl.ds(k * L, L)] = lax.shift_right_logical(v, 14)
        idb[pl.ds(k * L, L)] = lax.bitwise_and(v, PACK - 1)


def _zero_buf(buf, w):
    z = jnp.zeros((L,), jnp.float32)

    def zrow(r, _):
        for k in range(w // L):
            buf[r, pl.ds(k * L, L)] = z
        return 0

    lax.fori_loop(0, B, zrow, 0)


def _zero_rows(src_full, dst, base):
    for k in range(4):
        pltpu.sync_copy(src_full, dst.at[pl.ds(base + k * B, B)])
    pltpu.sync_copy(src_full.at[pl.ds(0, 120)],
                    dst.at[pl.ds(base + 4 * B, 120)])


DH = D // 2


def _make_gat_sc(heads, half, with_denom):
    mesh = plsc.VectorSubcoreMesh(core_axis_name="c", subcore_axis_name="s")

    def body(comb3d, asrc, adst, hw, *rest):
        if with_denom:
            numer_hbm, denom_hbm = rest[0], rest[1]
            scr = rest[2:]
        else:
            numer_hbm, denom_hbm = rest[0], None
            scr = rest[1:]
        (idx_c, isb0, idb0, ids0, isb1, idb1, ids1,
         ars0, ard0, hwb0, exb0, msg0,
         ars1, ard1, hwb1, exb1, msg1) = scr[:17]
        numer_sh = scr[17]
        if with_denom:
            denom_sh = scr[18]
            gsem0, gsem1, ssem0, ssem1 = scr[19:23]
        else:
            denom_sh = None
            gsem0, gsem1, ssem0, ssem1 = scr[18:22]
        cid = lax.axis_index("c")
        sid = lax.axis_index("s")
        wid = sid * NC + cid

        pltpu.sync_copy(comb3d.at[wid], idx_c)
        _zero_buf(msg0, DH)
        _zero_buf(exb0, L)
        _zero_rows(msg0, numer_sh, sid * RPT)
        if with_denom:
            _zero_rows(exb0, denom_sh, sid * RPT)
        plsc.subcore_barrier()

        bufs = ((isb0, idb0, ids0, ars0, ard0, hwb0, exb0, msg0, gsem0, ssem0),
                (isb1, idb1, ids1, ars1, ard1, hwb1, exb1, msg1, gsem1, ssem1))

        def start_gathers(p, c):
            isb, idb = bufs[p][0], bufs[p][1]
            ars, ard, hwb = bufs[p][3], bufs[p][4], bufs[p][5]
            gsem = bufs[p][8]
            _unpack_idx(idx_c, c, isb, idb)
            pltpu.async_copy(asrc.at[isb], ars, gsem)
            pltpu.async_copy(adst.at[idb], ard, gsem)
            pltpu.async_copy(hw.at[isb], hwb, gsem)

        start_gathers(0, 0)
        start_gathers(1, 1)

        def group(g, _):
            for p in range(2):
                c = g * 2 + p
                isb, idb, ids, ars, ard, hwb, exb, msgb, gsem, ssem = bufs[p]
                pltpu.make_async_copy(asrc.at[isb], ars, gsem).wait()
                pltpu.make_async_copy(adst.at[idb], ard, gsem).wait()
                pltpu.make_async_copy(hw.at[isb], hwb, gsem).wait()

                @pl.when(g >= 1)
                def _():
                    if with_denom:
                        pltpu.make_async_copy(exb, denom_sh.at[ids], ssem).wait()
                    pltpu.make_async_copy(msgb, numer_sh.at[ids], ssem).wait()

                def att(b, _):
                    e = ars[b] + ard[b]
                    e = jnp.maximum(e, 0.2 * e)
                    exb[b] = jnp.exp(e)
                    return 0

                def scale(b, _):
                    ex = exb[b]
                    if heads == 1:
                        sc = _splat(ex, 0)
                        for h in range(DH // L):
                            msgb[b, pl.ds(h * L, L)] = hwb[b, pl.ds(h * L, L)] * sc
                    else:
                        for h in range(DH // L):
                            sc = _splat(ex, half * (DH // L) + h)
                            msgb[b, pl.ds(h * L, L)] = hwb[b, pl.ds(h * L, L)] * sc
                    return 0

                lax.fori_loop(0, B, att, 0, unroll=4)
                lax.fori_loop(0, B, scale, 0, unroll=4)

                for k in range(B // L):
                    ids[pl.ds(k * L, L)] = idb[pl.ds(k * L, L)]
                if with_denom:
                    pltpu.async_copy(exb, denom_sh.at[ids], ssem, add=True)
                pltpu.async_copy(msgb, numer_sh.at[ids], ssem, add=True)

                cn = jnp.minimum(c + 2, CH - 1)
                start_gathers(p, cn)
            return 0

        lax.fori_loop(0, CH // 2, group, 0)

        for p in range(2):
            isb, idb, ids, ars, ard, hwb, exb, msgb, gsem, ssem = bufs[p]
            pltpu.make_async_copy(asrc.at[isb], ars, gsem).wait()
            pltpu.make_async_copy(adst.at[idb], ard, gsem).wait()
            pltpu.make_async_copy(hw.at[isb], hwb, gsem).wait()
            if with_denom:
                pltpu.make_async_copy(exb, denom_sh.at[ids], ssem).wait()
            pltpu.make_async_copy(msgb, numer_sh.at[ids], ssem).wait()

        plsc.subcore_barrier()
        pltpu.sync_copy(numer_sh.at[pl.ds(sid * RPT, RPT)],
                        numer_hbm.at[cid, pl.ds(sid * RPT, RPT)])
        if with_denom:
            pltpu.sync_copy(denom_sh.at[pl.ds(sid * RPT, RPT)],
                            denom_hbm.at[cid, pl.ds(sid * RPT, RPT)])

    out_type = [jax.ShapeDtypeStruct((NC, NPAD, DH), jnp.float32)]
    if with_denom:
        out_type.append(jax.ShapeDtypeStruct((NC, NPAD, L), jnp.float32))
    scratch = [
        pltpu.VMEM((CH, B), jnp.int32),
        pltpu.VMEM((B,), jnp.int32),
        pltpu.VMEM((B,), jnp.int32),
        pltpu.VMEM((B,), jnp.int32),
        pltpu.VMEM((B,), jnp.int32),
        pltpu.VMEM((B,), jnp.int32),
        pltpu.VMEM((B,), jnp.int32),
        pltpu.VMEM((B, L), jnp.float32),
        pltpu.VMEM((B, L), jnp.float32),
        pltpu.VMEM((B, DH), jnp.float32),
        pltpu.VMEM((B, L), jnp.float32),
        pltpu.VMEM((B, DH), jnp.float32),
        pltpu.VMEM((B, L), jnp.float32),
        pltpu.VMEM((B, L), jnp.float32),
        pltpu.VMEM((B, DH), jnp.float32),
        pltpu.VMEM((B, L), jnp.float32),
        pltpu.VMEM((B, DH), jnp.float32),
        pltpu.VMEM_SHARED((NPAD, DH), jnp.float32),
    ]
    if with_denom:
        scratch.append(pltpu.VMEM_SHARED((NPAD, L), jnp.float32))
    scratch += [pltpu.SemaphoreType.DMA] * 4

    return pl.kernel(
        body,
        out_type=tuple(out_type),
        mesh=mesh,
        compiler_params=pltpu.CompilerParams(use_tc_tiling_on_sc=False),
        scratch_types=scratch,
    )


def _make_pair_sc():
    mesh = plsc.VectorSubcoreMesh(core_axis_name="c", subcore_axis_name="s")

    def body(comb3d, ha, hb, g_hbm,
             idx_c, isb0, idb0, isb1, idb1,
             ra0, rb0, gr0, ra1, rb1, gr1,
             gsem0, gsem1, wsem0, wsem1):
        cid = lax.axis_index("c")
        sid = lax.axis_index("s")
        wid = sid * NC + cid
        pltpu.sync_copy(comb3d.at[wid], idx_c)

        bufs = ((isb0, idb0, ra0, rb0, gr0, gsem0, wsem0),
                (isb1, idb1, ra1, rb1, gr1, gsem1, wsem1))

        def start_gathers(p, c):
            isb, idb, ra, rb = bufs[p][0], bufs[p][1], bufs[p][2], bufs[p][3]
            gsem = bufs[p][5]
            _unpack_idx(idx_c, c, isb, idb)
            pltpu.async_copy(ha.at[isb], ra, gsem)
            pltpu.async_copy(hb.at[idb], rb, gsem)

        start_gathers(0, 0)
        start_gathers(1, 1)
        base = wid * CH0 * B

        def group(g, _):
            for p in range(2):
                c = g * 2 + p
                isb, idb, ra, rb, gr, gsem, wsem = bufs[p]
                pltpu.make_async_copy(ha.at[isb], ra, gsem).wait()
                pltpu.make_async_copy(hb.at[idb], rb, gsem).wait()

                @pl.when(g >= 1)
                def _():
                    pltpu.make_async_copy(
                        gr, g_hbm.at[pl.ds(base, B)], wsem).wait()

                def edge(b, _):
                    for h in range(D // L):
                        gr[b, pl.ds(h * L, L)] = (
                            ra[b, pl.ds(h * L, L)] + rb[b, pl.ds(h * L, L)])
                    return 0

                lax.fori_loop(0, B, edge, 0, unroll=4)
                pltpu.async_copy(gr, g_hbm.at[pl.ds(base + c * B, B)], wsem)
                cn = jnp.minimum(c + 2, CH0 - 1)
                start_gathers(p, cn)
            return 0

        lax.fori_loop(0, CH0 // 2, group, 0)
        for p in range(2):
            isb, idb, ra, rb, gr, gsem, wsem = bufs[p]
            pltpu.make_async_copy(ha.at[isb], ra, gsem).wait()
            pltpu.make_async_copy(hb.at[idb], rb, gsem).wait()
            pltpu.make_async_copy(gr, g_hbm.at[pl.ds(base, B)], wsem).wait()

    return pl.kernel(
        body,
        out_type=jax.ShapeDtypeStruct((EPAD0, D), jnp.float32),
        mesh=mesh,
        compiler_params=pltpu.CompilerParams(use_tc_tiling_on_sc=False),
        scratch_types=[
            pltpu.VMEM((CH0, B), jnp.int32),
            pltpu.VMEM((B,), jnp.int32),
            pltpu.VMEM((B,), jnp.int32),
            pltpu.VMEM((B,), jnp.int32),
            pltpu.VMEM((B,), jnp.int32),
            pltpu.VMEM((B, D), jnp.float32),
            pltpu.VMEM((B, D), jnp.float32),
            pltpu.VMEM((B, D), jnp.float32),
            pltpu.VMEM((B, D), jnp.float32),
            pltpu.VMEM((B, D), jnp.float32),
            pltpu.VMEM((B, D), jnp.float32),
            pltpu.SemaphoreType.DMA,
            pltpu.SemaphoreType.DMA,
            pltpu.SemaphoreType.DMA,
            pltpu.SemaphoreType.DMA,
        ],
    )


_gat8_a = _make_gat_sc(HEADS, 0, True)
_gat8_b = _make_gat_sc(HEADS, 1, False)
_gat1_a = _make_gat_sc(1, 0, True)
_gat1_b = _make_gat_sc(1, 1, False)
_pair_sc = _make_pair_sc()


def kernel(x, edge_index, W_in, b_in, g_in, bt_in, W1, as1, ad1, b1, g1, bt1,
           W2, as2, ad2, b2, g2, bt2, We1, be1, We2, be2, Wr1, br1, Wr2, br2):
    f32 = jnp.float32
    src0 = edge_index[0]
    dst0 = edge_index[1]
    loop = jnp.arange(N, dtype=src0.dtype)
    sent = jnp.full((EPAD - E,), N, src0.dtype)
    comb3d = (jnp.concatenate([src0, loop, sent]) * PACK
              + jnp.concatenate([dst0, loop, sent])).reshape(NW, CH, B)
    sent0 = jnp.full((EPAD0 - E0,), 0, src0.dtype)
    comb3d0 = (jnp.concatenate([src0, sent0]) * PACK
               + jnp.concatenate([dst0, sent0])).reshape(NW, CH0, B)

    hmask = (jnp.arange(D)[:, None] // L == jnp.arange(L)[None, :]).astype(f32)
    As1 = as1.reshape(D, 1) * hmask
    Ad1 = ad1.reshape(D, 1) * hmask
    As2 = jnp.pad(as2.reshape(D, 1), ((0, 0), (0, L - 1)))
    Ad2 = jnp.pad(ad2.reshape(D, 1), ((0, 0), (0, L - 1)))
    P1 = hmask.T
    P2 = jnp.pad(jnp.ones((1, D), f32), ((0, L - 1), (0, 0)))
    We2p = jnp.pad(We2, ((0, 0), (0, 32)))
    be2p = jnp.concatenate([be2, jnp.full((32,), -1e30, f32)]).reshape(1, 256)
    WrA, WrB = Wr1[:D], Wr1[D:]
    Wr2p = jnp.pad(Wr2, ((0, 0), (0, 2)))
    br2p = jnp.pad(br2, (0, 2)).reshape(1, 8)

    RB = 1000
    grid = (N // RB,)
    row = pl.BlockSpec((RB, D), lambda i: (i, 0))
    row16 = pl.BlockSpec((RB, L), lambda i: (i, 0))
    row2h = pl.BlockSpec((NC, RB, DH), lambda i: (0, i, 0))
    row216 = pl.BlockSpec((NC, RB, L), lambda i: (0, i, 0))
    vec = _full((1, D))

    def _one(r):
        return r[0] if isinstance(r, (tuple, list)) else r

    def _pad_tables(asr, adt, hw):
        asr_p = jnp.concatenate([asr, jnp.full((8, L), -1e30, f32)])
        adt_p = jnp.concatenate([adt, jnp.zeros((8, L), f32)])
        hw_l = jnp.concatenate([hw[:, :DH], jnp.zeros((8, DH), f32)])
        hw_r = jnp.concatenate([hw[:, DH:], jnp.zeros((8, DH), f32)])
        return asr_p, adt_p, hw_l, hw_r

    hw1, asr1, adt1 = pl.pallas_call(
        _tc_in_body,
        grid=grid,
        in_specs=[row, _full((D, D)), vec, vec, vec, _full((D, D)),
                  _full((D, L)), _full((D, L))],
        out_specs=[row, row16, row16],
        out_shape=[jax.ShapeDtypeStruct((N, D), f32),
                   jax.ShapeDtypeStruct((N, L), f32),
                   jax.ShapeDtypeStruct((N, L), f32)],
    )(x, W_in, b_in.reshape(1, D), g_in.reshape(1, D), bt_in.reshape(1, D),
      W1, As1, Ad1)

    asr1p, adt1p, hw1l, hw1r = _pad_tables(asr1, adt1, hw1)
    num1a, den1 = _gat8_a(comb3d, asr1p, adt1p, hw1l)
    num1b = _one(_gat8_b(comb3d, asr1p, adt1p, hw1r))

    hw2, asr2, adt2 = pl.pallas_call(
        _tc_mid_body,
        grid=grid,
        in_specs=[row2h, row2h, row216, _full((L, D)), vec, vec, vec,
                  _full((D, D)), _full((D, L)), _full((D, L))],
        out_specs=[row, row16, row16],
        out_shape=[jax.ShapeDtypeStruct((N, D), f32),
                   jax.ShapeDtypeStruct((N, L), f32),
                   jax.ShapeDtypeStruct((N, L), f32)],
    )(num1a, num1b, den1, P1, b1.reshape(1, D), g1.reshape(1, D),
      bt1.reshape(1, D), W2, As2, Ad2)

    asr2p, adt2p, hw2l, hw2r = _pad_tables(asr2, adt2, hw2)
    num2a, den2 = _gat1_a(comb3d, asr2p, adt2p, hw2l)
    num2b = _one(_gat1_b(comb3d, asr2p, adt2p, hw2r))

    entp, hA, hB = pl.pallas_call(
        _tc_out_body,
        grid=grid,
        in_specs=[row2h, row2h, row216, _full((L, D)), vec, vec, vec,
                  _full((D, D)), vec, _full((D, 256)), _full((1, 256)),
                  _full((D, D)), _full((D, D))],
        out_specs=[pl.BlockSpec((RB, 256), lambda i: (i, 0)), row, row],
        out_shape=[jax.ShapeDtypeStruct((N, 256), f32),
                   jax.ShapeDtypeStruct((N, D), f32),
                   jax.ShapeDtypeStruct((N, D), f32)],
    )(num2a, num2b, den2, P2, b2.reshape(1, D), g2.reshape(1, D), bt2.reshape(1, D),
      We1, be1.reshape(1, D), We2p, be2p, WrA, WrB)

    g = _pair_sc(comb3d0, hA, hB)

    EB = 4096
    relp = pl.pallas_call(
        _tc_rel_body,
        grid=(EPAD0 // EB,),
        in_specs=[pl.BlockSpec((EB, D), lambda i: (i, 0)), vec,
                  _full((D, 8)), _full((1, 8))],
        out_specs=pl.BlockSpec((EB, 8), lambda i: (i, 0)),
        out_shape=jax.ShapeDtypeStruct((EPAD0, 8), f32),
    )(g, br1.reshape(1, D), Wr2p, br2p)

    ent = entp[:, :224]
    rel = relp[:E0, :6]
    return ent, rel


# speedup vs baseline: 1.0140x; 1.0140x over previous
import jax
import jax.numpy as jnp
from jax import lax
from jax.experimental import pallas as pl
from jax.experimental.pallas import tpu as pltpu
from jax.experimental.pallas import tpu_sc as plsc

N = 10000
E0 = 320000
E = E0 + N
D = 128
HEADS = 8
NC, NS, L = 2, 16, 16
NW = NC * NS
B = 128
CH = 82
EPAD = NW * B * CH
CH0 = 80
EPAD0 = NW * B * CH0
RPT = 632
NPAD = NS * RPT
PACK = 16384


def _ln(x, g, b):
    m = jnp.mean(x, axis=-1, keepdims=True)
    c = x - m
    v = jnp.mean(c * c, axis=-1, keepdims=True)
    return c * lax.rsqrt(v + 1e-5) * g + b


def _elu(x):
    return jnp.where(x > 0, x, jnp.exp(jnp.minimum(x, 0.0)) - 1.0)


def _tc_in_body(x_ref, win, bin_, gin, btin, w1, as_, ad_, hw_ref, asr_ref, adt_ref):
    h0 = jnp.dot(x_ref[...], win[...], preferred_element_type=jnp.float32) + bin_[...]
    h = _ln(h0, gin[...], btin[...])
    hw = jnp.dot(h, w1[...], preferred_element_type=jnp.float32)
    hw_ref[...] = hw
    asr_ref[...] = jnp.dot(hw, as_[...], preferred_element_type=jnp.float32)
    adt_ref[...] = jnp.dot(hw, ad_[...], preferred_element_type=jnp.float32)


def _tc_mid_body(numa_ref, numb_ref, den_ref, p_ref, b_ref, g_ref, bt_ref,
                 w2, as_, ad_, hw_ref, asr_ref, adt_ref):
    num = jnp.concatenate([numa_ref[0] + numa_ref[1],
                           numb_ref[0] + numb_ref[1]], axis=-1)
    den = den_ref[0] + den_ref[1]
    denf = jnp.dot(den, p_ref[...], preferred_element_type=jnp.float32)
    gat = num / denf + b_ref[...]
    h = _elu(_ln(gat, g_ref[...], bt_ref[...]))
    hw = jnp.dot(h, w2[...], preferred_element_type=jnp.float32)
    hw_ref[...] = hw
    asr_ref[...] = jnp.dot(hw, as_[...], preferred_element_type=jnp.float32)
    adt_ref[...] = jnp.dot(hw, ad_[...], preferred_element_type=jnp.float32)


def _tc_out_body(numa_ref, numb_ref, den_ref, p_ref, b_ref, g_ref, bt_ref,
                 we1, be1, we2p, be2p, wra, wrb,
                 ent_ref, ha_ref, hb_ref):
    num = jnp.concatenate([numa_ref[0] + numa_ref[1],
                           numb_ref[0] + numb_ref[1]], axis=-1)
    den = den_ref[0] + den_ref[1]
    denf = jnp.dot(den, p_ref[...], preferred_element_type=jnp.float32)
    gat = num / denf + b_ref[...]
    h = _elu(_ln(gat, g_ref[...], bt_ref[...]))
    t = jnp.maximum(jnp.dot(h, we1[...], preferred_element_type=jnp.float32) + be1[...], 0.0)
    logits = jnp.dot(t, we2p[...], preferred_element_type=jnp.float32) + be2p[...]
    m = jnp.max(logits, axis=-1, keepdims=True)
    ex = jnp.exp(logits - m)
    sm = ex / jnp.sum(ex, axis=-1, keepdims=True)
    ent_ref[...] = jnp.log(sm + 1e-8)
    ha_ref[...] = jnp.dot(h, wra[...], preferred_element_type=jnp.float32)
    hb_ref[...] = jnp.dot(h, wrb[...], preferred_element_type=jnp.float32)


def _tc_rel_body(g_ref, br1_ref, wr2_ref, br2_ref, rel_ref):
    t = jnp.maximum(g_ref[...].astype(jnp.float32) + br1_ref[...], 0.0)
    rel_ref[...] = jnp.dot(t, wr2_ref[...], preferred_element_type=jnp.float32) + br2_ref[...]


def _full(shape):
    return pl.BlockSpec(shape, lambda i: tuple(0 for _ in shape))


def _splat(v, lane):
    return lax.gather(
        v, jnp.full((L, 1), lane, jnp.int32),
        dimension_numbers=lax.GatherDimensionNumbers(
            offset_dims=(), collapsed_slice_dims=(0,), start_index_map=(0,)),
        slice_sizes=(1,),
        mode=lax.GatherScatterMode.PROMISE_IN_BOUNDS)


def _unpack_idx(idx_c, c, isb, idb):
    for k in range(B // L):
        v = idx_c[c, pl.ds(k * L, L)]
        isb[pl.ds(k * L, L)] = lax.shift_right_logical(v, 14)
        idb[pl.ds(k * L, L)] = lax.bitwise_and(v, PACK - 1)


def _zero_buf(buf, w):
    z = jnp.zeros((L,), jnp.float32)

    def zrow(r, _):
        for k in range(w // L):
            buf[r, pl.ds(k * L, L)] = z
        return 0

    lax.fori_loop(0, B, zrow, 0)


def _zero_rows(src_full, dst, base):
    for k in range(4):
        pltpu.sync_copy(src_full, dst.at[pl.ds(base + k * B, B)])
    pltpu.sync_copy(src_full.at[pl.ds(0, 120)],
                    dst.at[pl.ds(base + 4 * B, 120)])


DH = D // 2


def _make_gat_sc(heads, half, with_denom):
    mesh = plsc.VectorSubcoreMesh(core_axis_name="c", subcore_axis_name="s")

    def body(comb3d, asrc, adst, hw, *rest):
        if with_denom:
            numer_hbm, denom_hbm = rest[0], rest[1]
            scr = rest[2:]
        else:
            numer_hbm, denom_hbm = rest[0], None
            scr = rest[1:]
        (idx_c, isb0, idb0, ids0, isb1, idb1, ids1,
         ars0, ard0, hwb0, exb0, msg0,
         ars1, ard1, hwb1, exb1, msg1) = scr[:17]
        numer_sh = scr[17]
        if with_denom:
            denom_sh = scr[18]
            gsem0, gsem1, ssem0, ssem1 = scr[19:23]
        else:
            denom_sh = None
            gsem0, gsem1, ssem0, ssem1 = scr[18:22]
        cid = lax.axis_index("c")
        sid = lax.axis_index("s")
        wid = sid * NC + cid

        pltpu.sync_copy(comb3d.at[wid], idx_c)
        _zero_buf(msg0, DH)
        _zero_buf(exb0, L)
        _zero_rows(msg0, numer_sh, sid * RPT)
        if with_denom:
            _zero_rows(exb0, denom_sh, sid * RPT)
        plsc.subcore_barrier()

        bufs = ((isb0, idb0, ids0, ars0, ard0, hwb0, exb0, msg0, gsem0, ssem0),
                (isb1, idb1, ids1, ars1, ard1, hwb1, exb1, msg1, gsem1, ssem1))

        def start_gathers(p, c):
            isb, idb = bufs[p][0], bufs[p][1]
            ars, ard, hwb = bufs[p][3], bufs[p][4], bufs[p][5]
            gsem = bufs[p][8]
            _unpack_idx(idx_c, c, isb, idb)
            pltpu.async_copy(asrc.at[isb], ars, gsem)
            pltpu.async_copy(adst.at[idb], ard, gsem)
            pltpu.async_copy(hw.at[isb], hwb, gsem)

        start_gathers(0, 0)
        start_gathers(1, 1)

        def group(g, _):
            for p in range(2):
                c = g * 2 + p
                isb, idb, ids, ars, ard, hwb, exb, msgb, gsem, ssem = bufs[p]
                pltpu.make_async_copy(asrc.at[isb], ars, gsem).wait()
                pltpu.make_async_copy(adst.at[idb], ard, gsem).wait()
                pltpu.make_async_copy(hw.at[isb], hwb, gsem).wait()

                @pl.when(g >= 1)
                def _():
                    if with_denom:
                        pltpu.make_async_copy(exb, denom_sh.at[ids], ssem).wait()
                    pltpu.make_async_copy(msgb, numer_sh.at[ids], ssem).wait()

                def edge(b, _):
                    e = ars[b] + ard[b]
                    e = jnp.maximum(e, 0.2 * e)
                    ex = jnp.exp(e)
                    if with_denom:
                        exb[b] = ex
                    if heads == 1:
                        sc = _splat(ex, 0)
                        for h in range(DH // L):
                            msgb[b, pl.ds(h * L, L)] = hwb[b, pl.ds(h * L, L)] * sc
                    else:
                        for h in range(DH // L):
                            sc = _splat(ex, half * (DH // L) + h)
                            msgb[b, pl.ds(h * L, L)] = hwb[b, pl.ds(h * L, L)] * sc
                    return 0

                lax.fori_loop(0, B, edge, 0, unroll=2)

                for k in range(B // L):
                    ids[pl.ds(k * L, L)] = idb[pl.ds(k * L, L)]
                if with_denom:
                    pltpu.async_copy(exb, denom_sh.at[ids], ssem, add=True)
                pltpu.async_copy(msgb, numer_sh.at[ids], ssem, add=True)

                cn = jnp.minimum(c + 2, CH - 1)
                start_gathers(p, cn)
            return 0

        lax.fori_loop(0, CH // 2, group, 0)

        for p in range(2):
            isb, idb, ids, ars, ard, hwb, exb, msgb, gsem, ssem = bufs[p]
            pltpu.make_async_copy(asrc.at[isb], ars, gsem).wait()
            pltpu.make_async_copy(adst.at[idb], ard, gsem).wait()
            pltpu.make_async_copy(hw.at[isb], hwb, gsem).wait()
            if with_denom:
                pltpu.make_async_copy(exb, denom_sh.at[ids], ssem).wait()
            pltpu.make_async_copy(msgb, numer_sh.at[ids], ssem).wait()

        plsc.subcore_barrier()
        pltpu.sync_copy(numer_sh.at[pl.ds(sid * RPT, RPT)],
                        numer_hbm.at[cid, pl.ds(sid * RPT, RPT)])
        if with_denom:
            pltpu.sync_copy(denom_sh.at[pl.ds(sid * RPT, RPT)],
                            denom_hbm.at[cid, pl.ds(sid * RPT, RPT)])

    out_type = [jax.ShapeDtypeStruct((NC, NPAD, DH), jnp.float32)]
    if with_denom:
        out_type.append(jax.ShapeDtypeStruct((NC, NPAD, L), jnp.float32))
    scratch = [
        pltpu.VMEM((CH, B), jnp.int32),
        pltpu.VMEM((B,), jnp.int32),
        pltpu.VMEM((B,), jnp.int32),
        pltpu.VMEM((B,), jnp.int32),
        pltpu.VMEM((B,), jnp.int32),
        pltpu.VMEM((B,), jnp.int32),
        pltpu.VMEM((B,), jnp.int32),
        pltpu.VMEM((B, L), jnp.float32),
        pltpu.VMEM((B, L), jnp.float32),
        pltpu.VMEM((B, DH), jnp.float32),
        pltpu.VMEM((B, L), jnp.float32),
        pltpu.VMEM((B, DH), jnp.float32),
        pltpu.VMEM((B, L), jnp.float32),
        pltpu.VMEM((B, L), jnp.float32),
        pltpu.VMEM((B, DH), jnp.float32),
        pltpu.VMEM((B, L), jnp.float32),
        pltpu.VMEM((B, DH), jnp.float32),
        pltpu.VMEM_SHARED((NPAD, DH), jnp.float32),
    ]
    if with_denom:
        scratch.append(pltpu.VMEM_SHARED((NPAD, L), jnp.float32))
    scratch += [pltpu.SemaphoreType.DMA] * 4

    return pl.kernel(
        body,
        out_type=tuple(out_type),
        mesh=mesh,
        compiler_params=pltpu.CompilerParams(use_tc_tiling_on_sc=False),
        scratch_types=scratch,
    )


def _make_pair_sc():
    mesh = plsc.VectorSubcoreMesh(core_axis_name="c", subcore_axis_name="s")

    def body(comb3d, ha, hb, g_hbm,
             idx_c, isb0, idb0, isb1, idb1,
             ra0, rb0, gr0, ra1, rb1, gr1,
             gsem0, gsem1, wsem0, wsem1):
        cid = lax.axis_index("c")
        sid = lax.axis_index("s")
        wid = sid * NC + cid
        pltpu.sync_copy(comb3d.at[wid], idx_c)

        bufs = ((isb0, idb0, ra0, rb0, gr0, gsem0, wsem0),
                (isb1, idb1, ra1, rb1, gr1, gsem1, wsem1))

        def start_gathers(p, c):
            isb, idb, ra, rb = bufs[p][0], bufs[p][1], bufs[p][2], bufs[p][3]
            gsem = bufs[p][5]
            _unpack_idx(idx_c, c, isb, idb)
            pltpu.async_copy(ha.at[isb], ra, gsem)
            pltpu.async_copy(hb.at[idb], rb, gsem)

        start_gathers(0, 0)
        start_gathers(1, 1)
        base = wid * CH0 * B

        def group(g, _):
            for p in range(2):
                c = g * 2 + p
                isb, idb, ra, rb, gr, gsem, wsem = bufs[p]
                pltpu.make_async_copy(ha.at[isb], ra, gsem).wait()
                pltpu.make_async_copy(hb.at[idb], rb, gsem).wait()

                @pl.when(g >= 1)
                def _():
                    pltpu.make_async_copy(
                        gr, g_hbm.at[pl.ds(base, B)], wsem).wait()

                def edge(b, _):
                    for h in range(D // 32):
                        gr[b, pl.ds(h * 32, 32)] = (
                            ra[b, pl.ds(h * 32, 32)] + rb[b, pl.ds(h * 32, 32)])
                    return 0

                lax.fori_loop(0, B, edge, 0, unroll=4)
                pltpu.async_copy(gr, g_hbm.at[pl.ds(base + c * B, B)], wsem)
                cn = jnp.minimum(c + 2, CH0 - 1)
                start_gathers(p, cn)
            return 0

        lax.fori_loop(0, CH0 // 2, group, 0)
        for p in range(2):
            isb, idb, ra, rb, gr, gsem, wsem = bufs[p]
            pltpu.make_async_copy(ha.at[isb], ra, gsem).wait()
            pltpu.make_async_copy(hb.at[idb], rb, gsem).wait()
            pltpu.make_async_copy(gr, g_hbm.at[pl.ds(base, B)], wsem).wait()

    return pl.kernel(
        body,
        out_type=jax.ShapeDtypeStruct((EPAD0, D), jnp.bfloat16),
        mesh=mesh,
        compiler_params=pltpu.CompilerParams(use_tc_tiling_on_sc=False),
        scratch_types=[
            pltpu.VMEM((CH0, B), jnp.int32),
            pltpu.VMEM((B,), jnp.int32),
            pltpu.VMEM((B,), jnp.int32),
            pltpu.VMEM((B,), jnp.int32),
            pltpu.VMEM((B,), jnp.int32),
            pltpu.VMEM((B, D), jnp.bfloat16),
            pltpu.VMEM((B, D), jnp.bfloat16),
            pltpu.VMEM((B, D), jnp.bfloat16),
            pltpu.VMEM((B, D), jnp.bfloat16),
            pltpu.VMEM((B, D), jnp.bfloat16),
            pltpu.VMEM((B, D), jnp.bfloat16),
            pltpu.SemaphoreType.DMA,
            pltpu.SemaphoreType.DMA,
            pltpu.SemaphoreType.DMA,
            pltpu.SemaphoreType.DMA,
        ],
    )


_gat8_a = _make_gat_sc(HEADS, 0, True)
_gat8_b = _make_gat_sc(HEADS, 1, False)
_gat1_a = _make_gat_sc(1, 0, True)
_gat1_b = _make_gat_sc(1, 1, False)
_pair_sc = _make_pair_sc()


def kernel(x, edge_index, W_in, b_in, g_in, bt_in, W1, as1, ad1, b1, g1, bt1,
           W2, as2, ad2, b2, g2, bt2, We1, be1, We2, be2, Wr1, br1, Wr2, br2):
    f32 = jnp.float32
    src0 = edge_index[0]
    dst0 = edge_index[1]
    loop = jnp.arange(N, dtype=src0.dtype)
    sent = jnp.full((EPAD - E,), N, src0.dtype)
    comb3d = (jnp.concatenate([src0, loop, sent]) * PACK
              + jnp.concatenate([dst0, loop, sent])).reshape(NW, CH, B)
    sent0 = jnp.full((EPAD0 - E0,), 0, src0.dtype)
    comb3d0 = (jnp.concatenate([src0, sent0]) * PACK
               + jnp.concatenate([dst0, sent0])).reshape(NW, CH0, B)

    hmask = (jnp.arange(D)[:, None] // L == jnp.arange(L)[None, :]).astype(f32)
    As1 = as1.reshape(D, 1) * hmask
    Ad1 = ad1.reshape(D, 1) * hmask
    As2 = jnp.pad(as2.reshape(D, 1), ((0, 0), (0, L - 1)))
    Ad2 = jnp.pad(ad2.reshape(D, 1), ((0, 0), (0, L - 1)))
    P1 = hmask.T
    P2 = jnp.pad(jnp.ones((1, D), f32), ((0, L - 1), (0, 0)))
    We2p = jnp.pad(We2, ((0, 0), (0, 32)))
    be2p = jnp.concatenate([be2, jnp.full((32,), -1e30, f32)]).reshape(1, 256)
    WrA, WrB = Wr1[:D], Wr1[D:]
    Wr2p = jnp.pad(Wr2, ((0, 0), (0, 2)))
    br2p = jnp.pad(br2, (0, 2)).reshape(1, 8)

    RB = 1000
    grid = (N // RB,)
    row = pl.BlockSpec((RB, D), lambda i: (i, 0))
    row16 = pl.BlockSpec((RB, L), lambda i: (i, 0))
    row2h = pl.BlockSpec((NC, RB, DH), lambda i: (0, i, 0))
    row216 = pl.BlockSpec((NC, RB, L), lambda i: (0, i, 0))
    vec = _full((1, D))

    def _one(r):
        return r[0] if isinstance(r, (tuple, list)) else r

    def _pad_tables(asr, adt, hw):
        asr_p = jnp.concatenate([asr, jnp.full((8, L), -1e30, f32)])
        adt_p = jnp.concatenate([adt, jnp.zeros((8, L), f32)])
        hw_l = jnp.concatenate([hw[:, :DH], jnp.zeros((8, DH), f32)])
        hw_r = jnp.concatenate([hw[:, DH:], jnp.zeros((8, DH), f32)])
        return asr_p, adt_p, hw_l, hw_r

    hw1, asr1, adt1 = pl.pallas_call(
        _tc_in_body,
        grid=grid,
        in_specs=[row, _full((D, D)), vec, vec, vec, _full((D, D)),
                  _full((D, L)), _full((D, L))],
        out_specs=[row, row16, row16],
        out_shape=[jax.ShapeDtypeStruct((N, D), f32),
                   jax.ShapeDtypeStruct((N, L), f32),
                   jax.ShapeDtypeStruct((N, L), f32)],
    )(x, W_in, b_in.reshape(1, D), g_in.reshape(1, D), bt_in.reshape(1, D),
      W1, As1, Ad1)

    asr1p, adt1p, hw1l, hw1r = _pad_tables(asr1, adt1, hw1)
    num1a, den1 = _gat8_a(comb3d, asr1p, adt1p, hw1l)
    num1b = _one(_gat8_b(comb3d, asr1p, adt1p, hw1r))

    hw2, asr2, adt2 = pl.pallas_call(
        _tc_mid_body,
        grid=grid,
        in_specs=[row2h, row2h, row216, _full((L, D)), vec, vec, vec,
                  _full((D, D)), _full((D, L)), _full((D, L))],
        out_specs=[row, row16, row16],
        out_shape=[jax.ShapeDtypeStruct((N, D), f32),
                   jax.ShapeDtypeStruct((N, L), f32),
                   jax.ShapeDtypeStruct((N, L), f32)],
    )(num1a, num1b, den1, P1, b1.reshape(1, D), g1.reshape(1, D),
      bt1.reshape(1, D), W2, As2, Ad2)

    asr2p, adt2p, hw2l, hw2r = _pad_tables(asr2, adt2, hw2)
    num2a, den2 = _gat1_a(comb3d, asr2p, adt2p, hw2l)
    num2b = _one(_gat1_b(comb3d, asr2p, adt2p, hw2r))

    entp, hA, hB = pl.pallas_call(
        _tc_out_body,
        grid=grid,
        in_specs=[row2h, row2h, row216, _full((L, D)), vec, vec, vec,
                  _full((D, D)), vec, _full((D, 256)), _full((1, 256)),
                  _full((D, D)), _full((D, D))],
        out_specs=[pl.BlockSpec((RB, 256), lambda i: (i, 0)), row, row],
        out_shape=[jax.ShapeDtypeStruct((N, 256), f32),
                   jax.ShapeDtypeStruct((N, D), f32),
                   jax.ShapeDtypeStruct((N, D), f32)],
    )(num2a, num2b, den2, P2, b2.reshape(1, D), g2.reshape(1, D), bt2.reshape(1, D),
      We1, be1.reshape(1, D), We2p, be2p, WrA, WrB)

    g = _pair_sc(comb3d0, hA.astype(jnp.bfloat16), hB.astype(jnp.bfloat16))

    EB = 4096
    relp = pl.pallas_call(
        _tc_rel_body,
        grid=(EPAD0 // EB,),
        in_specs=[pl.BlockSpec((EB, D), lambda i: (i, 0)), vec,
                  _full((D, 8)), _full((1, 8))],
        out_specs=pl.BlockSpec((EB, 8), lambda i: (i, 0)),
        out_shape=jax.ShapeDtypeStruct((EPAD0, 8), f32),
    )(g, br1.reshape(1, D), Wr2p, br2p)

    ent = entp[:, :224]
    rel = relp[:E0, :6]
    return ent, rel


# speedup vs baseline: 1.0354x; 1.0211x over previous
import jax
import jax.numpy as jnp
from jax import lax
from jax.experimental import pallas as pl
from jax.experimental.pallas import tpu as pltpu
from jax.experimental.pallas import tpu_sc as plsc

N = 10000
E0 = 320000
E = E0 + N
D = 128
HEADS = 8
NC, NS, L = 2, 16, 16
NW = NC * NS
B = 128
CH = 82
EPAD = NW * B * CH
CH0 = 80
EPAD0 = NW * B * CH0
RPT = 632
NPAD = NS * RPT
PACK = 16384


def _ln(x, g, b):
    m = jnp.mean(x, axis=-1, keepdims=True)
    c = x - m
    v = jnp.mean(c * c, axis=-1, keepdims=True)
    return c * lax.rsqrt(v + 1e-5) * g + b


def _elu(x):
    return jnp.where(x > 0, x, jnp.exp(jnp.minimum(x, 0.0)) - 1.0)


def _tc_in_body(x_ref, win, bin_, gin, btin, w1, as_, ad_, hw_ref, asr_ref, adt_ref):
    h0 = jnp.dot(x_ref[...], win[...], preferred_element_type=jnp.float32) + bin_[...]
    h = _ln(h0, gin[...], btin[...])
    hw = jnp.dot(h, w1[...], preferred_element_type=jnp.float32)
    hw_ref[...] = hw
    asr_ref[...] = jnp.dot(hw, as_[...], preferred_element_type=jnp.float32)
    adt_ref[...] = jnp.dot(hw, ad_[...], preferred_element_type=jnp.float32)


def _tc_mid_body(numa_ref, numb_ref, den_ref, p_ref, b_ref, g_ref, bt_ref,
                 w2, as_, ad_, hw_ref, asr_ref, adt_ref):
    num = jnp.concatenate([numa_ref[0] + numa_ref[1],
                           numb_ref[0] + numb_ref[1]], axis=-1)
    den = den_ref[0] + den_ref[1]
    denf = jnp.dot(den, p_ref[...], preferred_element_type=jnp.float32)
    gat = num / denf + b_ref[...]
    h = _elu(_ln(gat, g_ref[...], bt_ref[...]))
    hw = jnp.dot(h, w2[...], preferred_element_type=jnp.float32)
    hw_ref[...] = hw
    asr_ref[...] = jnp.dot(hw, as_[...], preferred_element_type=jnp.float32)
    adt_ref[...] = jnp.dot(hw, ad_[...], preferred_element_type=jnp.float32)


def _tc_out_body(numa_ref, numb_ref, den_ref, p_ref, b_ref, g_ref, bt_ref,
                 we1, be1, we2p, be2p, wra, wrb,
                 ent_ref, ha_ref, hb_ref):
    num = jnp.concatenate([numa_ref[0] + numa_ref[1],
                           numb_ref[0] + numb_ref[1]], axis=-1)
    den = den_ref[0] + den_ref[1]
    denf = jnp.dot(den, p_ref[...], preferred_element_type=jnp.float32)
    gat = num / denf + b_ref[...]
    h = _elu(_ln(gat, g_ref[...], bt_ref[...]))
    t = jnp.maximum(jnp.dot(h, we1[...], preferred_element_type=jnp.float32) + be1[...], 0.0)
    logits = jnp.dot(t, we2p[...], preferred_element_type=jnp.float32) + be2p[...]
    m = jnp.max(logits, axis=-1, keepdims=True)
    ex = jnp.exp(logits - m)
    sm = ex / jnp.sum(ex, axis=-1, keepdims=True)
    ent_ref[...] = jnp.log(sm + 1e-8)
    ha_ref[...] = jnp.dot(h, wra[...], preferred_element_type=jnp.float32)
    hb_ref[...] = jnp.dot(h, wrb[...], preferred_element_type=jnp.float32)


def _tc_rel_body(g_ref, br1_ref, wr2_ref, br2_ref, rel_ref):
    t = jnp.maximum(g_ref[...] + br1_ref[...], 0.0)
    rel_ref[...] = jnp.dot(t, wr2_ref[...], preferred_element_type=jnp.float32) + br2_ref[...]


def _full(shape):
    return pl.BlockSpec(shape, lambda i: tuple(0 for _ in shape))


def _splat(v, lane):
    return lax.gather(
        v, jnp.full((L, 1), lane, jnp.int32),
        dimension_numbers=lax.GatherDimensionNumbers(
            offset_dims=(), collapsed_slice_dims=(0,), start_index_map=(0,)),
        slice_sizes=(1,),
        mode=lax.GatherScatterMode.PROMISE_IN_BOUNDS)


def _unpack_idx(idx_c, c, isb, idb):
    for k in range(B // L):
        v = idx_c[c, pl.ds(k * L, L)]
        isb[pl.ds(k * L, L)] = lax.shift_right_logical(v, 14)
        idb[pl.ds(k * L, L)] = lax.bitwise_and(v, PACK - 1)


def _zero_buf(buf, w):
    z = jnp.zeros((L,), jnp.float32)

    def zrow(r, _):
        for k in range(w // L):
            buf[r, pl.ds(k * L, L)] = z
        return 0

    lax.fori_loop(0, B, zrow, 0)


def _zero_rows(src_full, dst, base):
    for k in range(4):
        pltpu.sync_copy(src_full, dst.at[pl.ds(base + k * B, B)])
    pltpu.sync_copy(src_full.at[pl.ds(0, 120)],
                    dst.at[pl.ds(base + 4 * B, 120)])


DH = D // 2


def _make_gat_sc(heads, half, with_denom):
    mesh = plsc.VectorSubcoreMesh(core_axis_name="c", subcore_axis_name="s")

    def body(comb3d, asrc, adst, hw, *rest):
        if with_denom:
            numer_hbm, denom_hbm = rest[0], rest[1]
            scr = rest[2:]
        else:
            numer_hbm, denom_hbm = rest[0], None
            scr = rest[1:]
        (idx_c, isb0, idb0, ids0, isb1, idb1, ids1,
         ars0, ard0, hwb0, exb0, msg0,
         ars1, ard1, hwb1, exb1, msg1) = scr[:17]
        numer_sh = scr[17]
        if with_denom:
            denom_sh = scr[18]
            gsem0, gsem1, ssem0, ssem1 = scr[19:23]
        else:
            denom_sh = None
            gsem0, gsem1, ssem0, ssem1 = scr[18:22]
        cid = lax.axis_index("c")
        sid = lax.axis_index("s")
        wid = sid * NC + cid

        pltpu.sync_copy(comb3d.at[wid], idx_c)
        _zero_buf(msg0, DH)
        _zero_buf(exb0, L)
        _zero_rows(msg0, numer_sh, sid * RPT)
        if with_denom:
            _zero_rows(exb0, denom_sh, sid * RPT)
        plsc.subcore_barrier()

        bufs = ((isb0, idb0, ids0, ars0, ard0, hwb0, exb0, msg0, gsem0, ssem0),
                (isb1, idb1, ids1, ars1, ard1, hwb1, exb1, msg1, gsem1, ssem1))

        def start_gathers(p, c):
            isb, idb = bufs[p][0], bufs[p][1]
            ars, ard, hwb = bufs[p][3], bufs[p][4], bufs[p][5]
            gsem = bufs[p][8]
            _unpack_idx(idx_c, c, isb, idb)
            pltpu.async_copy(asrc.at[isb], ars, gsem)
            pltpu.async_copy(adst.at[idb], ard, gsem)
            pltpu.async_copy(hw.at[isb], hwb, gsem)

        start_gathers(0, 0)
        start_gathers(1, 1)

        def group(g, _):
            for p in range(2):
                c = g * 2 + p
                isb, idb, ids, ars, ard, hwb, exb, msgb, gsem, ssem = bufs[p]
                pltpu.make_async_copy(asrc.at[isb], ars, gsem).wait()
                pltpu.make_async_copy(adst.at[idb], ard, gsem).wait()
                pltpu.make_async_copy(hw.at[isb], hwb, gsem).wait()

                @pl.when(g >= 1)
                def _():
                    if with_denom:
                        pltpu.make_async_copy(exb, denom_sh.at[ids], ssem).wait()
                    pltpu.make_async_copy(msgb, numer_sh.at[ids], ssem).wait()

                def edge(b, _):
                    e = ars[b] + ard[b]
                    e = jnp.maximum(e, 0.2 * e)
                    ex = jnp.exp(e)
                    if with_denom:
                        exb[b] = ex
                    if heads == 1:
                        sc = _splat(ex, 0)
                        for h in range(DH // L):
                            msgb[b, pl.ds(h * L, L)] = hwb[b, pl.ds(h * L, L)] * sc
                    else:
                        for h in range(DH // L):
                            sc = _splat(ex, half * (DH // L) + h)
                            msgb[b, pl.ds(h * L, L)] = hwb[b, pl.ds(h * L, L)] * sc
                    return 0

                lax.fori_loop(0, B, edge, 0, unroll=2)

                for k in range(B // L):
                    ids[pl.ds(k * L, L)] = idb[pl.ds(k * L, L)]
                if with_denom:
                    pltpu.async_copy(exb, denom_sh.at[ids], ssem, add=True)
                pltpu.async_copy(msgb, numer_sh.at[ids], ssem, add=True)

                cn = jnp.minimum(c + 2, CH - 1)
                start_gathers(p, cn)
            return 0

        lax.fori_loop(0, CH // 2, group, 0)

        for p in range(2):
            isb, idb, ids, ars, ard, hwb, exb, msgb, gsem, ssem = bufs[p]
            pltpu.make_async_copy(asrc.at[isb], ars, gsem).wait()
            pltpu.make_async_copy(adst.at[idb], ard, gsem).wait()
            pltpu.make_async_copy(hw.at[isb], hwb, gsem).wait()
            if with_denom:
                pltpu.make_async_copy(exb, denom_sh.at[ids], ssem).wait()
            pltpu.make_async_copy(msgb, numer_sh.at[ids], ssem).wait()

        plsc.subcore_barrier()
        pltpu.sync_copy(numer_sh.at[pl.ds(sid * RPT, RPT)],
                        numer_hbm.at[cid, pl.ds(sid * RPT, RPT)])
        if with_denom:
            pltpu.sync_copy(denom_sh.at[pl.ds(sid * RPT, RPT)],
                            denom_hbm.at[cid, pl.ds(sid * RPT, RPT)])

    out_type = [jax.ShapeDtypeStruct((NC, NPAD, DH), jnp.float32)]
    if with_denom:
        out_type.append(jax.ShapeDtypeStruct((NC, NPAD, L), jnp.float32))
    scratch = [
        pltpu.VMEM((CH, B), jnp.int32),
        pltpu.VMEM((B,), jnp.int32),
        pltpu.VMEM((B,), jnp.int32),
        pltpu.VMEM((B,), jnp.int32),
        pltpu.VMEM((B,), jnp.int32),
        pltpu.VMEM((B,), jnp.int32),
        pltpu.VMEM((B,), jnp.int32),
        pltpu.VMEM((B, L), jnp.float32),
        pltpu.VMEM((B, L), jnp.float32),
        pltpu.VMEM((B, DH), jnp.float32),
        pltpu.VMEM((B, L), jnp.float32),
        pltpu.VMEM((B, DH), jnp.float32),
        pltpu.VMEM((B, L), jnp.float32),
        pltpu.VMEM((B, L), jnp.float32),
        pltpu.VMEM((B, DH), jnp.float32),
        pltpu.VMEM((B, L), jnp.float32),
        pltpu.VMEM((B, DH), jnp.float32),
        pltpu.VMEM_SHARED((NPAD, DH), jnp.float32),
    ]
    if with_denom:
        scratch.append(pltpu.VMEM_SHARED((NPAD, L), jnp.float32))
    scratch += [pltpu.SemaphoreType.DMA] * 4

    return pl.kernel(
        body,
        out_type=tuple(out_type),
        mesh=mesh,
        compiler_params=pltpu.CompilerParams(use_tc_tiling_on_sc=False),
        scratch_types=scratch,
    )


def _make_pair_sc():
    mesh = plsc.VectorSubcoreMesh(core_axis_name="c", subcore_axis_name="s")

    def body(comb3d, ha, hb, g_hbm,
             idx_c, isb0, idb0, isb1, idb1,
             ra0, rb0, gr0, ra1, rb1, gr1,
             gsem0, gsem1, wsem0, wsem1):
        cid = lax.axis_index("c")
        sid = lax.axis_index("s")
        wid = sid * NC + cid
        pltpu.sync_copy(comb3d.at[wid], idx_c)

        bufs = ((isb0, idb0, ra0, rb0, gr0, gsem0, wsem0),
                (isb1, idb1, ra1, rb1, gr1, gsem1, wsem1))

        def start_gathers(p, c):
            isb, idb, ra, rb = bufs[p][0], bufs[p][1], bufs[p][2], bufs[p][3]
            gsem = bufs[p][5]
            _unpack_idx(idx_c, c, isb, idb)
            pltpu.async_copy(ha.at[isb], ra, gsem)
            pltpu.async_copy(hb.at[idb], rb, gsem)

        start_gathers(0, 0)
        start_gathers(1, 1)
        base = wid * CH0 * B

        def group(g, _):
            for p in range(2):
                c = g * 2 + p
                isb, idb, ra, rb, gr, gsem, wsem = bufs[p]
                pltpu.make_async_copy(ha.at[isb], ra, gsem).wait()
                pltpu.make_async_copy(hb.at[idb], rb, gsem).wait()

                @pl.when(g >= 1)
                def _():
                    pltpu.make_async_copy(
                        gr, g_hbm.at[pl.ds(base, B)], wsem).wait()

                def edge(b, _):
                    for h in range(D // L):
                        gr[b, pl.ds(h * L, L)] = (
                            ra[b, pl.ds(h * L, L)] + rb[b, pl.ds(h * L, L)])
                    return 0

                lax.fori_loop(0, B, edge, 0, unroll=2)
                pltpu.async_copy(gr, g_hbm.at[pl.ds(base + c * B, B)], wsem)
                cn = jnp.minimum(c + 2, CH0 - 1)
                start_gathers(p, cn)
            return 0

        lax.fori_loop(0, CH0 // 2, group, 0)
        for p in range(2):
            isb, idb, ra, rb, gr, gsem, wsem = bufs[p]
            pltpu.make_async_copy(ha.at[isb], ra, gsem).wait()
            pltpu.make_async_copy(hb.at[idb], rb, gsem).wait()
            pltpu.make_async_copy(gr, g_hbm.at[pl.ds(base, B)], wsem).wait()

    return pl.kernel(
        body,
        out_type=jax.ShapeDtypeStruct((EPAD0, D), jnp.float32),
        mesh=mesh,
        compiler_params=pltpu.CompilerParams(use_tc_tiling_on_sc=False),
        scratch_types=[
            pltpu.VMEM((CH0, B), jnp.int32),
            pltpu.VMEM((B,), jnp.int32),
            pltpu.VMEM((B,), jnp.int32),
            pltpu.VMEM((B,), jnp.int32),
            pltpu.VMEM((B,), jnp.int32),
            pltpu.VMEM((B, D), jnp.float32),
            pltpu.VMEM((B, D), jnp.float32),
            pltpu.VMEM((B, D), jnp.float32),
            pltpu.VMEM((B, D), jnp.float32),
            pltpu.VMEM((B, D), jnp.float32),
            pltpu.VMEM((B, D), jnp.float32),
            pltpu.SemaphoreType.DMA,
            pltpu.SemaphoreType.DMA,
            pltpu.SemaphoreType.DMA,
            pltpu.SemaphoreType.DMA,
        ],
    )


_gat8_a = _make_gat_sc(HEADS, 0, True)
_gat8_b = _make_gat_sc(HEADS, 1, False)
_gat1_a = _make_gat_sc(1, 0, True)
_gat1_b = _make_gat_sc(1, 1, False)
_pair_sc = _make_pair_sc()


def kernel(x, edge_index, W_in, b_in, g_in, bt_in, W1, as1, ad1, b1, g1, bt1,
           W2, as2, ad2, b2, g2, bt2, We1, be1, We2, be2, Wr1, br1, Wr2, br2):
    f32 = jnp.float32
    src0 = edge_index[0]
    dst0 = edge_index[1]
    loop = jnp.arange(N, dtype=src0.dtype)
    sent = jnp.full((EPAD - E,), N, src0.dtype)
    comb3d = (jnp.concatenate([src0, loop, sent]) * PACK
              + jnp.concatenate([dst0, loop, sent])).reshape(NW, CH, B)
    sent0 = jnp.full((EPAD0 - E0,), 0, src0.dtype)
    comb3d0 = (jnp.concatenate([src0, sent0]) * PACK
               + jnp.concatenate([dst0, sent0])).reshape(NW, CH0, B)

    hmask = (jnp.arange(D)[:, None] // L == jnp.arange(L)[None, :]).astype(f32)
    As1 = as1.reshape(D, 1) * hmask
    Ad1 = ad1.reshape(D, 1) * hmask
    As2 = jnp.pad(as2.reshape(D, 1), ((0, 0), (0, L - 1)))
    Ad2 = jnp.pad(ad2.reshape(D, 1), ((0, 0), (0, L - 1)))
    P1 = hmask.T
    P2 = jnp.pad(jnp.ones((1, D), f32), ((0, L - 1), (0, 0)))
    We2p = jnp.pad(We2, ((0, 0), (0, 32)))
    be2p = jnp.concatenate([be2, jnp.full((32,), -1e30, f32)]).reshape(1, 256)
    WrA, WrB = Wr1[:D], Wr1[D:]
    Wr2p = jnp.pad(Wr2, ((0, 0), (0, 2)))
    br2p = jnp.pad(br2, (0, 2)).reshape(1, 8)

    RB = 1000
    grid = (N // RB,)
    row = pl.BlockSpec((RB, D), lambda i: (i, 0))
    row16 = pl.BlockSpec((RB, L), lambda i: (i, 0))
    row2h = pl.BlockSpec((NC, RB, DH), lambda i: (0, i, 0))
    row216 = pl.BlockSpec((NC, RB, L), lambda i: (0, i, 0))
    vec = _full((1, D))

    def _one(r):
        return r[0] if isinstance(r, (tuple, list)) else r

    def _pad_tables(asr, adt, hw):
        asr_p = jnp.concatenate([asr, jnp.full((8, L), -1e30, f32)])
        adt_p = jnp.concatenate([adt, jnp.zeros((8, L), f32)])
        hw_l = jnp.concatenate([hw[:, :DH], jnp.zeros((8, DH), f32)])
        hw_r = jnp.concatenate([hw[:, DH:], jnp.zeros((8, DH), f32)])
        return asr_p, adt_p, hw_l, hw_r

    hw1, asr1, adt1 = pl.pallas_call(
        _tc_in_body,
        grid=grid,
        in_specs=[row, _full((D, D)), vec, vec, vec, _full((D, D)),
                  _full((D, L)), _full((D, L))],
        out_specs=[row, row16, row16],
        out_shape=[jax.ShapeDtypeStruct((N, D), f32),
                   jax.ShapeDtypeStruct((N, L), f32),
                   jax.ShapeDtypeStruct((N, L), f32)],
    )(x, W_in, b_in.reshape(1, D), g_in.reshape(1, D), bt_in.reshape(1, D),
      W1, As1, Ad1)

    asr1p, adt1p, hw1l, hw1r = _pad_tables(asr1, adt1, hw1)
    num1a, den1 = _gat8_a(comb3d, asr1p, adt1p, hw1l)
    num1b = _one(_gat8_b(comb3d, asr1p, adt1p, hw1r))

    hw2, asr2, adt2 = pl.pallas_call(
        _tc_mid_body,
        grid=grid,
        in_specs=[row2h, row2h, row216, _full((L, D)), vec, vec, vec,
                  _full((D, D)), _full((D, L)), _full((D, L))],
        out_specs=[row, row16, row16],
        out_shape=[jax.ShapeDtypeStruct((N, D), f32),
                   jax.ShapeDtypeStruct((N, L), f32),
                   jax.ShapeDtypeStruct((N, L), f32)],
    )(num1a, num1b, den1, P1, b1.reshape(1, D), g1.reshape(1, D),
      bt1.reshape(1, D), W2, As2, Ad2)

    asr2p, adt2p, hw2l, hw2r = _pad_tables(asr2, adt2, hw2)
    num2a, den2 = _gat1_a(comb3d, asr2p, adt2p, hw2l)
    num2b = _one(_gat1_b(comb3d, asr2p, adt2p, hw2r))

    entp, hA, hB = pl.pallas_call(
        _tc_out_body,
        grid=grid,
        in_specs=[row2h, row2h, row216, _full((L, D)), vec, vec, vec,
                  _full((D, D)), vec, _full((D, 256)), _full((1, 256)),
                  _full((D, D)), _full((D, D))],
        out_specs=[pl.BlockSpec((RB, 256), lambda i: (i, 0)), row, row],
        out_shape=[jax.ShapeDtypeStruct((N, 256), f32),
                   jax.ShapeDtypeStruct((N, D), f32),
                   jax.ShapeDtypeStruct((N, D), f32)],
    )(num2a, num2b, den2, P2, b2.reshape(1, D), g2.reshape(1, D), bt2.reshape(1, D),
      We1, be1.reshape(1, D), We2p, be2p, WrA, WrB)

    g = _pair_sc(comb3d0, hA, hB)

    EB = 4096
    relp = pl.pallas_call(
        _tc_rel_body,
        grid=(EPAD0 // EB,),
        in_specs=[pl.BlockSpec((EB, D), lambda i: (i, 0)), vec,
                  _full((D, 8)), _full((1, 8))],
        out_specs=pl.BlockSpec((EB, 8), lambda i: (i, 0)),
        out_shape=jax.ShapeDtypeStruct((EPAD0, 8), f32),
    )(g, br1.reshape(1, D), Wr2p, br2p)

    ent = entp[:, :224]
    rel = relp[:E0, :6]
    return ent, rel


# speedup vs baseline: 1.0358x; 1.0004x over previous
import jax
import jax.numpy as jnp
from jax import lax
from jax.experimental import pallas as pl
from jax.experimental.pallas import tpu as pltpu
from jax.experimental.pallas import tpu_sc as plsc

N = 10000
E0 = 320000
E = E0 + N
D = 128
HEADS = 8
NC, NS, L = 2, 16, 16
NW = NC * NS
B = 128
CH = 82
EPAD = NW * B * CH
CH0 = 80
EPAD0 = NW * B * CH0
RPT = 632
NPAD = NS * RPT
PACK = 16384


def _ln(x, g, b):
    m = jnp.mean(x, axis=-1, keepdims=True)
    c = x - m
    v = jnp.mean(c * c, axis=-1, keepdims=True)
    return c * lax.rsqrt(v + 1e-5) * g + b


def _elu(x):
    return jnp.where(x > 0, x, jnp.exp(jnp.minimum(x, 0.0)) - 1.0)


def _tc_in_body(x_ref, win, bin_, gin, btin, w1, as_, ad_, hw_ref, asr_ref, adt_ref):
    h0 = jnp.dot(x_ref[...], win[...], preferred_element_type=jnp.float32) + bin_[...]
    h = _ln(h0, gin[...], btin[...])
    hw = jnp.dot(h, w1[...], preferred_element_type=jnp.float32)
    hw_ref[...] = hw
    asr_ref[...] = jnp.dot(hw, as_[...], preferred_element_type=jnp.float32)
    adt_ref[...] = jnp.dot(hw, ad_[...], preferred_element_type=jnp.float32)


def _tc_mid_body(numa_ref, numb_ref, den_ref, p_ref, b_ref, g_ref, bt_ref,
                 w2, as_, ad_, hw_ref, asr_ref, adt_ref):
    num = jnp.concatenate([numa_ref[0] + numa_ref[1],
                           numb_ref[0] + numb_ref[1]], axis=-1)
    den = den_ref[0] + den_ref[1]
    denf = jnp.dot(den, p_ref[...], preferred_element_type=jnp.float32)
    gat = num / denf + b_ref[...]
    h = _elu(_ln(gat, g_ref[...], bt_ref[...]))
    hw = jnp.dot(h, w2[...], preferred_element_type=jnp.float32)
    hw_ref[...] = hw
    asr_ref[...] = jnp.dot(hw, as_[...], preferred_element_type=jnp.float32)
    adt_ref[...] = jnp.dot(hw, ad_[...], preferred_element_type=jnp.float32)


def _tc_out_body(numa_ref, numb_ref, den_ref, p_ref, b_ref, g_ref, bt_ref,
                 we1, be1, we2p, be2p, wra, wrb,
                 ent_ref, ha_ref, hb_ref):
    num = jnp.concatenate([numa_ref[0] + numa_ref[1],
                           numb_ref[0] + numb_ref[1]], axis=-1)
    den = den_ref[0] + den_ref[1]
    denf = jnp.dot(den, p_ref[...], preferred_element_type=jnp.float32)
    gat = num / denf + b_ref[...]
    h = _elu(_ln(gat, g_ref[...], bt_ref[...]))
    t = jnp.maximum(jnp.dot(h, we1[...], preferred_element_type=jnp.float32) + be1[...], 0.0)
    logits = jnp.dot(t, we2p[...], preferred_element_type=jnp.float32) + be2p[...]
    m = jnp.max(logits, axis=-1, keepdims=True)
    ex = jnp.exp(logits - m)
    sm = ex / jnp.sum(ex, axis=-1, keepdims=True)
    ent_ref[...] = jnp.log(sm + 1e-8)
    ha_ref[...] = jnp.dot(h, wra[...], preferred_element_type=jnp.float32)
    hb_ref[...] = jnp.dot(h, wrb[...], preferred_element_type=jnp.float32)


def _tc_rel_body(g_ref, br1_ref, wr2_ref, br2_ref, rel_ref):
    t = jnp.maximum(g_ref[...] + br1_ref[...], 0.0)
    rel_ref[...] = jnp.dot(t, wr2_ref[...], preferred_element_type=jnp.float32) + br2_ref[...]


def _full(shape):
    return pl.BlockSpec(shape, lambda i: tuple(0 for _ in shape))


def _splat(v, lane):
    return lax.gather(
        v, jnp.full((L, 1), lane, jnp.int32),
        dimension_numbers=lax.GatherDimensionNumbers(
            offset_dims=(), collapsed_slice_dims=(0,), start_index_map=(0,)),
        slice_sizes=(1,),
        mode=lax.GatherScatterMode.PROMISE_IN_BOUNDS)


def _unpack_idx(idx_c, c, isb, idb):
    for k in range(B // L):
        v = idx_c[c, pl.ds(k * L, L)]
        isb[pl.ds(k * L, L)] = lax.shift_right_logical(v, 14)
        idb[pl.ds(k * L, L)] = lax.bitwise_and(v, PACK - 1)


def _zero_buf(buf, w):
    z = jnp.zeros((L,), jnp.float32)

    def zrow(r, _):
        for k in range(w // L):
            buf[r, pl.ds(k * L, L)] = z
        return 0

    lax.fori_loop(0, B, zrow, 0)


def _zero_rows(src_full, dst, base):
    for k in range(4):
        pltpu.sync_copy(src_full, dst.at[pl.ds(base + k * B, B)])
    pltpu.sync_copy(src_full.at[pl.ds(0, 120)],
                    dst.at[pl.ds(base + 4 * B, 120)])


DH = D // 2


def _make_gat_sc(heads, half, with_denom):
    mesh = plsc.VectorSubcoreMesh(core_axis_name="c", subcore_axis_name="s")

    def body(comb3d, asrc, adst, hw, *rest):
        if with_denom:
            numer_hbm, denom_hbm = rest[0], rest[1]
            scr = rest[2:]
        else:
            numer_hbm, denom_hbm = rest[0], None
            scr = rest[1:]
        (idx_c, isb0, idb0, ids0, isb1, idb1, ids1,
         ars0, ard0, hwb0, exb0, msg0,
         ars1, ard1, hwb1, exb1, msg1) = scr[:17]
        numer_sh = scr[17]
        if with_denom:
            denom_sh = scr[18]
            gsem0, gsem1, ssem0, ssem1 = scr[19:23]
        else:
            denom_sh = None
            gsem0, gsem1, ssem0, ssem1 = scr[18:22]
        cid = lax.axis_index("c")
        sid = lax.axis_index("s")
        wid = sid * NC + cid

        pltpu.sync_copy(comb3d.at[wid], idx_c)
        _zero_buf(msg0, DH)
        _zero_buf(exb0, L)
        _zero_rows(msg0, numer_sh, sid * RPT)
        if with_denom:
            _zero_rows(exb0, denom_sh, sid * RPT)
        plsc.subcore_barrier()

        bufs = ((isb0, idb0, ids0, ars0, ard0, hwb0, exb0, msg0, gsem0, ssem0),
                (isb1, idb1, ids1, ars1, ard1, hwb1, exb1, msg1, gsem1, ssem1))

        def start_gathers(p, c):
            isb, idb = bufs[p][0], bufs[p][1]
            ars, ard, hwb = bufs[p][3], bufs[p][4], bufs[p][5]
            gsem = bufs[p][8]
            _unpack_idx(idx_c, c, isb, idb)
            pltpu.async_copy(asrc.at[isb], ars, gsem)
            pltpu.async_copy(adst.at[idb], ard, gsem)
            pltpu.async_copy(hw.at[isb], hwb, gsem)

        start_gathers(0, 0)
        start_gathers(1, 1)

        def group(g, _):
            for p in range(2):
                c = g * 2 + p
                isb, idb, ids, ars, ard, hwb, exb, msgb, gsem, ssem = bufs[p]
                pltpu.make_async_copy(asrc.at[isb], ars, gsem).wait()
                pltpu.make_async_copy(adst.at[idb], ard, gsem).wait()
                pltpu.make_async_copy(hw.at[isb], hwb, gsem).wait()

                @pl.when(g >= 1)
                def _():
                    if with_denom:
                        pltpu.make_async_copy(exb, denom_sh.at[ids], ssem).wait()
                    pltpu.make_async_copy(msgb, numer_sh.at[ids], ssem).wait()

                def edge(b, _):
                    e = ars[b] + ard[b]
                    e = jnp.maximum(e, 0.2 * e)
                    ex = jnp.exp(e)
                    if with_denom:
                        exb[b] = ex
                    if heads == 1:
                        sc = _splat(ex, 0)
                        for h in range(DH // L):
                            msgb[b, pl.ds(h * L, L)] = hwb[b, pl.ds(h * L, L)] * sc
                    else:
                        for h in range(DH // L):
                            sc = _splat(ex, half * (DH // L) + h)
                            msgb[b, pl.ds(h * L, L)] = hwb[b, pl.ds(h * L, L)] * sc
                    return 0

                lax.fori_loop(0, B, edge, 0, unroll=4)

                for k in range(B // L):
                    ids[pl.ds(k * L, L)] = idb[pl.ds(k * L, L)]
                if with_denom:
                    pltpu.async_copy(exb, denom_sh.at[ids], ssem, add=True)
                pltpu.async_copy(msgb, numer_sh.at[ids], ssem, add=True)

                cn = jnp.minimum(c + 2, CH - 1)
                start_gathers(p, cn)
            return 0

        lax.fori_loop(0, CH // 2, group, 0)

        for p in range(2):
            isb, idb, ids, ars, ard, hwb, exb, msgb, gsem, ssem = bufs[p]
            pltpu.make_async_copy(asrc.at[isb], ars, gsem).wait()
            pltpu.make_async_copy(adst.at[idb], ard, gsem).wait()
            pltpu.make_async_copy(hw.at[isb], hwb, gsem).wait()
            if with_denom:
                pltpu.make_async_copy(exb, denom_sh.at[ids], ssem).wait()
            pltpu.make_async_copy(msgb, numer_sh.at[ids], ssem).wait()

        plsc.subcore_barrier()
        pltpu.sync_copy(numer_sh.at[pl.ds(sid * RPT, RPT)],
                        numer_hbm.at[cid, pl.ds(sid * RPT, RPT)])
        if with_denom:
            pltpu.sync_copy(denom_sh.at[pl.ds(sid * RPT, RPT)],
                            denom_hbm.at[cid, pl.ds(sid * RPT, RPT)])

    out_type = [jax.ShapeDtypeStruct((NC, NPAD, DH), jnp.float32)]
    if with_denom:
        out_type.append(jax.ShapeDtypeStruct((NC, NPAD, L), jnp.float32))
    scratch = [
        pltpu.VMEM((CH, B), jnp.int32),
        pltpu.VMEM((B,), jnp.int32),
        pltpu.VMEM((B,), jnp.int32),
        pltpu.VMEM((B,), jnp.int32),
        pltpu.VMEM((B,), jnp.int32),
        pltpu.VMEM((B,), jnp.int32),
        pltpu.VMEM((B,), jnp.int32),
        pltpu.VMEM((B, L), jnp.float32),
        pltpu.VMEM((B, L), jnp.float32),
        pltpu.VMEM((B, DH), jnp.float32),
        pltpu.VMEM((B, L), jnp.float32),
        pltpu.VMEM((B, DH), jnp.float32),
        pltpu.VMEM((B, L), jnp.float32),
        pltpu.VMEM((B, L), jnp.float32),
        pltpu.VMEM((B, DH), jnp.float32),
        pltpu.VMEM((B, L), jnp.float32),
        pltpu.VMEM((B, DH), jnp.float32),
        pltpu.VMEM_SHARED((NPAD, DH), jnp.float32),
    ]
    if with_denom:
        scratch.append(pltpu.VMEM_SHARED((NPAD, L), jnp.float32))
    scratch += [pltpu.SemaphoreType.DMA] * 4

    return pl.kernel(
        body,
        out_type=tuple(out_type),
        mesh=mesh,
        compiler_params=pltpu.CompilerParams(use_tc_tiling_on_sc=False),
        scratch_types=scratch,
    )


def _make_pair_sc():
    mesh = plsc.VectorSubcoreMesh(core_axis_name="c", subcore_axis_name="s")

    def body(comb3d, ha, hb, g_hbm,
             idx_c, isb0, idb0, isb1, idb1,
             ra0, rb0, gr0, ra1, rb1, gr1,
             gsem0, gsem1, wsem0, wsem1):
        cid = lax.axis_index("c")
        sid = lax.axis_index("s")
        wid = sid * NC + cid
        pltpu.sync_copy(comb3d.at[wid], idx_c)

        bufs = ((isb0, idb0, ra0, rb0, gr0, gsem0, wsem0),
                (isb1, idb1, ra1, rb1, gr1, gsem1, wsem1))

        def start_gathers(p, c):
            isb, idb, ra, rb = bufs[p][0], bufs[p][1], bufs[p][2], bufs[p][3]
            gsem = bufs[p][5]
            _unpack_idx(idx_c, c, isb, idb)
            pltpu.async_copy(ha.at[isb], ra, gsem)
            pltpu.async_copy(hb.at[idb], rb, gsem)

        start_gathers(0, 0)
        start_gathers(1, 1)
        base = wid * CH0 * B

        def group(g, _):
            for p in range(2):
                c = g * 2 + p
                isb, idb, ra, rb, gr, gsem, wsem = bufs[p]
                pltpu.make_async_copy(ha.at[isb], ra, gsem).wait()
                pltpu.make_async_copy(hb.at[idb], rb, gsem).wait()

                @pl.when(g >= 1)
                def _():
                    pltpu.make_async_copy(
                        gr, g_hbm.at[pl.ds(base, B)], wsem).wait()

                def edge(b, _):
                    for h in range(D // L):
                        gr[b, pl.ds(h * L, L)] = (
                            ra[b, pl.ds(h * L, L)] + rb[b, pl.ds(h * L, L)])
                    return 0

                lax.fori_loop(0, B, edge, 0, unroll=4)
                pltpu.async_copy(gr, g_hbm.at[pl.ds(base + c * B, B)], wsem)
                cn = jnp.minimum(c + 2, CH0 - 1)
                start_gathers(p, cn)
            return 0

        lax.fori_loop(0, CH0 // 2, group, 0)
        for p in range(2):
            isb, idb, ra, rb, gr, gsem, wsem = bufs[p]
            pltpu.make_async_copy(ha.at[isb], ra, gsem).wait()
            pltpu.make_async_copy(hb.at[idb], rb, gsem).wait()
            pltpu.make_async_copy(gr, g_hbm.at[pl.ds(base, B)], wsem).wait()

    return pl.kernel(
        body,
        out_type=jax.ShapeDtypeStruct((EPAD0, D), jnp.float32),
        mesh=mesh,
        compiler_params=pltpu.CompilerParams(use_tc_tiling_on_sc=False),
        scratch_types=[
            pltpu.VMEM((CH0, B), jnp.int32),
            pltpu.VMEM((B,), jnp.int32),
            pltpu.VMEM((B,), jnp.int32),
            pltpu.VMEM((B,), jnp.int32),
            pltpu.VMEM((B,), jnp.int32),
            pltpu.VMEM((B, D), jnp.float32),
            pltpu.VMEM((B, D), jnp.float32),
            pltpu.VMEM((B, D), jnp.float32),
            pltpu.VMEM((B, D), jnp.float32),
            pltpu.VMEM((B, D), jnp.float32),
            pltpu.VMEM((B, D), jnp.float32),
            pltpu.SemaphoreType.DMA,
            pltpu.SemaphoreType.DMA,
            pltpu.SemaphoreType.DMA,
            pltpu.SemaphoreType.DMA,
        ],
    )


_gat8_a = _make_gat_sc(HEADS, 0, True)
_gat8_b = _make_gat_sc(HEADS, 1, False)
_gat1_a = _make_gat_sc(1, 0, True)
_gat1_b = _make_gat_sc(1, 1, False)
_pair_sc = _make_pair_sc()


def kernel(x, edge_index, W_in, b_in, g_in, bt_in, W1, as1, ad1, b1, g1, bt1,
           W2, as2, ad2, b2, g2, bt2, We1, be1, We2, be2, Wr1, br1, Wr2, br2):
    f32 = jnp.float32
    src0 = edge_index[0]
    dst0 = edge_index[1]
    loop = jnp.arange(N, dtype=src0.dtype)
    sent = jnp.full((EPAD - E,), N, src0.dtype)
    comb3d = (jnp.concatenate([src0, loop, sent]) * PACK
              + jnp.concatenate([dst0, loop, sent])).reshape(NW, CH, B)
    sent0 = jnp.full((EPAD0 - E0,), 0, src0.dtype)
    comb3d0 = (jnp.concatenate([src0, sent0]) * PACK
               + jnp.concatenate([dst0, sent0])).reshape(NW, CH0, B)

    hmask = (jnp.arange(D)[:, None] // L == jnp.arange(L)[None, :]).astype(f32)
    As1 = as1.reshape(D, 1) * hmask
    Ad1 = ad1.reshape(D, 1) * hmask
    As2 = jnp.pad(as2.reshape(D, 1), ((0, 0), (0, L - 1)))
    Ad2 = jnp.pad(ad2.reshape(D, 1), ((0, 0), (0, L - 1)))
    P1 = hmask.T
    P2 = jnp.pad(jnp.ones((1, D), f32), ((0, L - 1), (0, 0)))
    We2p = jnp.pad(We2, ((0, 0), (0, 32)))
    be2p = jnp.concatenate([be2, jnp.full((32,), -1e30, f32)]).reshape(1, 256)
    WrA, WrB = Wr1[:D], Wr1[D:]
    Wr2p = jnp.pad(Wr2, ((0, 0), (0, 2)))
    br2p = jnp.pad(br2, (0, 2)).reshape(1, 8)

    RB = 1000
    grid = (N // RB,)
    row = pl.BlockSpec((RB, D), lambda i: (i, 0))
    row16 = pl.BlockSpec((RB, L), lambda i: (i, 0))
    row2h = pl.BlockSpec((NC, RB, DH), lambda i: (0, i, 0))
    row216 = pl.BlockSpec((NC, RB, L), lambda i: (0, i, 0))
    vec = _full((1, D))

    def _one(r):
        return r[0] if isinstance(r, (tuple, list)) else r

    def _pad_tables(asr, adt, hw):
        asr_p = jnp.concatenate([asr, jnp.full((8, L), -1e30, f32)])
        adt_p = jnp.concatenate([adt, jnp.zeros((8, L), f32)])
        hw_l = jnp.concatenate([hw[:, :DH], jnp.zeros((8, DH), f32)])
        hw_r = jnp.concatenate([hw[:, DH:], jnp.zeros((8, DH), f32)])
        return asr_p, adt_p, hw_l, hw_r

    hw1, asr1, adt1 = pl.pallas_call(
        _tc_in_body,
        grid=grid,
        in_specs=[row, _full((D, D)), vec, vec, vec, _full((D, D)),
                  _full((D, L)), _full((D, L))],
        out_specs=[row, row16, row16],
        out_shape=[jax.ShapeDtypeStruct((N, D), f32),
                   jax.ShapeDtypeStruct((N, L), f32),
                   jax.ShapeDtypeStruct((N, L), f32)],
    )(x, W_in, b_in.reshape(1, D), g_in.reshape(1, D), bt_in.reshape(1, D),
      W1, As1, Ad1)

    asr1p, adt1p, hw1l, hw1r = _pad_tables(asr1, adt1, hw1)
    num1a, den1 = _gat8_a(comb3d, asr1p, adt1p, hw1l)
    num1b = _one(_gat8_b(comb3d, asr1p, adt1p, hw1r))

    hw2, asr2, adt2 = pl.pallas_call(
        _tc_mid_body,
        grid=grid,
        in_specs=[row2h, row2h, row216, _full((L, D)), vec, vec, vec,
                  _full((D, D)), _full((D, L)), _full((D, L))],
        out_specs=[row, row16, row16],
        out_shape=[jax.ShapeDtypeStruct((N, D), f32),
                   jax.ShapeDtypeStruct((N, L), f32),
                   jax.ShapeDtypeStruct((N, L), f32)],
    )(num1a, num1b, den1, P1, b1.reshape(1, D), g1.reshape(1, D),
      bt1.reshape(1, D), W2, As2, Ad2)

    asr2p, adt2p, hw2l, hw2r = _pad_tables(asr2, adt2, hw2)
    num2a, den2 = _gat1_a(comb3d, asr2p, adt2p, hw2l)
    num2b = _one(_gat1_b(comb3d, asr2p, adt2p, hw2r))

    entp, hA, hB = pl.pallas_call(
        _tc_out_body,
        grid=grid,
        in_specs=[row2h, row2h, row216, _full((L, D)), vec, vec, vec,
                  _full((D, D)), vec, _full((D, 256)), _full((1, 256)),
                  _full((D, D)), _full((D, D))],
        out_specs=[pl.BlockSpec((RB, 256), lambda i: (i, 0)), row, row],
        out_shape=[jax.ShapeDtypeStruct((N, 256), f32),
                   jax.ShapeDtypeStruct((N, D), f32),
                   jax.ShapeDtypeStruct((N, D), f32)],
    )(num2a, num2b, den2, P2, b2.reshape(1, D), g2.reshape(1, D), bt2.reshape(1, D),
      We1, be1.reshape(1, D), We2p, be2p, WrA, WrB)

    g = _pair_sc(comb3d0, hA, hB)

    EB = 4096
    relp = pl.pallas_call(
        _tc_rel_body,
        grid=(EPAD0 // EB,),
        in_specs=[pl.BlockSpec((EB, D), lambda i: (i, 0)), vec,
                  _full((D, 8)), _full((1, 8))],
        out_specs=pl.BlockSpec((EB, 8), lambda i: (i, 0)),
        out_shape=jax.ShapeDtypeStruct((EPAD0, 8), f32),
    )(g, br1.reshape(1, D), Wr2p, br2p)

    ent = entp[:, :224]
    rel = relp[:E0, :6]
    return ent, rel
